# baseline (speedup 1.0000x reference)
import math

import jax
import jax.numpy as jnp
from jax.experimental import pallas as pl

_SIZE = 32000
_PAD = 0
_SMOOTH = 0.1
_CONF = 1.0 - _SMOOTH
_EPS = _SMOOTH / (_SIZE - 2)
_N = 4096
_BM = 128
_BN = _SIZE // 2
_C1 = _EPS * math.log(_EPS) * (_SIZE - 2) + _CONF * math.log(_CONF)


def _kl_kernel(t_ref, xa_ref, xb_ref, o_ref):
    i = pl.program_id(0)

    @pl.when(i == 0)
    def _():
        o_ref[...] = jnp.zeros_like(o_ref)

    t = t_ref[...]
    live = t != _PAD
    cols = jax.lax.broadcasted_iota(jnp.int32, (_BM, _BN), 1)

    ya = jnp.where(cols == t, (_CONF / _EPS) * xa_ref[...], xa_ref[...])
    yb = jnp.where(cols + _BN == t, (_CONF / _EPS) * xb_ref[...], xb_ref[...])
    rowsum = jnp.sum(ya, axis=1, keepdims=True) + jnp.sum(yb, axis=1, keepdims=True)
    acc = jnp.sum(jnp.where(live, -_EPS, 0.0) * rowsum)

    livef = jnp.where(live, 1.0, 0.0)
    extra = jnp.sum(livef * (_EPS * xa_ref[:, 0:1] + _C1))

    o_ref[...] += (acc + extra).reshape(1, 1)


@jax.jit
def kernel(x, target):
    t32 = target.astype(jnp.int32).reshape(_N, 1)
    out = pl.pallas_call(
        _kl_kernel,
        grid=(_N // _BM,),
        in_specs=[
            pl.BlockSpec((_BM, 1), lambda i: (i, 0)),
            pl.BlockSpec((_BM, _BN), lambda i: (i, 0)),
            pl.BlockSpec((_BM, _BN), lambda i: (i, 1)),
        ],
        out_specs=pl.BlockSpec((1, 1), lambda i: (0, 0)),
        out_shape=jax.ShapeDtypeStruct((1, 1), jnp.float32),
    )(t32, x, x)
    return out[0, 0]
